# baseline (speedup 1.0000x reference)
import functools

import jax
import jax.numpy as jnp
from jax import lax
from jax.experimental import pallas as pl
from jax.experimental.pallas import tpu as pltpu
from jax.experimental.pallas import tpu_sc as plsc


def kernel(subset, embedding):
    (B,) = subset.shape
    V, D = embedding.shape
    L = 16

    view = embedding.reshape(V // 2, 2 * D)

    info = plsc.get_sparse_core_info()
    NC, NS = info.num_cores, info.num_subcores
    NW = NC * NS
    b_per_w = B // NW
    G = b_per_w // L

    mesh = plsc.VectorSubcoreMesh(core_axis_name="c", subcore_axis_name="s")

    @functools.partial(
        pl.kernel,
        mesh=mesh,
        out_type=jax.ShapeDtypeStruct((B * D,), jnp.float32),
        scratch_types=[
            pltpu.VMEM((b_per_w,), jnp.int32),
            pltpu.VMEM((b_per_w,), jnp.int32),
            pltpu.VMEM((b_per_w,), jnp.int32),
            pltpu.VMEM((b_per_w, 2 * D), jnp.float32),
            pltpu.VMEM((b_per_w * D,), jnp.float32),
            pltpu.SemaphoreType.DMA,
        ],
    )
    def gather_kernel(idx_hbm, view_hbm, out_hbm, idx_v, pair_v, off_v,
                      rows_v, out_v, sem):
        wid = lax.axis_index("s") * NC + lax.axis_index("c")
        base = wid * b_per_w
        pltpu.sync_copy(idx_hbm.at[pl.ds(base, b_per_w)], idx_v)

        def prep(g, carry):
            x = idx_v[pl.ds(g * L, L)]
            pair_v[pl.ds(g * L, L)] = lax.shift_right_logical(x, 1)
            off_v[pl.ds(g * L, L)] = lax.shift_left(
                lax.bitwise_and(x, 1), 6)
            return carry

        lax.fori_loop(0, G, prep, 0)

        pltpu.async_copy(view_hbm.at[pair_v], rows_v, sem).wait()

        def select(g, carry):
            off_vec = off_v[pl.ds(g * L, L)]
            for l in range(L):
                b = g * L + l
                off = off_vec[l]
                for j in range(D // L):
                    out_v[pl.ds(b * D + j * L, L)] = (
                        rows_v[b, pl.ds(off + j * L, L)])
            return carry

        lax.fori_loop(0, G, select, 0)

        pltpu.sync_copy(out_v, out_hbm.at[pl.ds(base * D, b_per_w * D)])

    out = gather_kernel(subset.astype(jnp.int32), view)
    return out.reshape(B, D)


# speedup vs baseline: 1.7516x; 1.7516x over previous
import functools

import jax
import jax.numpy as jnp
from jax import lax
from jax.experimental import pallas as pl
from jax.experimental.pallas import tpu as pltpu
from jax.experimental.pallas import tpu_sc as plsc


def kernel(subset, embedding):
    (B,) = subset.shape
    V, D = embedding.shape
    L = 16

    info = plsc.get_sparse_core_info()
    NC, NS = info.num_cores, info.num_subcores
    NW = NC * NS
    b_per_w = B // NW
    G = b_per_w // L

    mesh = plsc.VectorSubcoreMesh(core_axis_name="c", subcore_axis_name="s")

    @functools.partial(
        pl.kernel,
        mesh=mesh,
        out_type=jax.ShapeDtypeStruct((B, D), jnp.float32),
        scratch_types=[
            pltpu.VMEM((b_per_w,), jnp.int32),
            pltpu.VMEM((b_per_w, D), jnp.float32),
            pltpu.SemaphoreType.DMA,
        ],
    )
    def gather_kernel(idx_hbm, table_hbm, out_hbm, idx_v, rows_v, sem):
        wid = lax.axis_index("s") * NC + lax.axis_index("c")
        base = wid * b_per_w
        pltpu.sync_copy(idx_hbm.at[pl.ds(base, b_per_w)], idx_v)

        def issue(g, carry):
            iv = idx_v[pl.ds(g * L, L)]
            for l in range(L):
                pltpu.async_copy(table_hbm.at[iv[l]],
                                 rows_v.at[g * L + l], sem)
            return carry

        lax.fori_loop(0, G, issue, 0)

        pltpu.make_async_copy(table_hbm.at[pl.ds(0, b_per_w)],
                              rows_v, sem).wait()

        pltpu.sync_copy(rows_v, out_hbm.at[pl.ds(base, b_per_w)])

    return gather_kernel(subset.astype(jnp.int32), embedding)
